# baseline (speedup 1.0000x reference)
import functools

import jax
import jax.numpy as jnp
from jax.experimental import pallas as pl
from jax.experimental.pallas import tpu as pltpu


def _se_gate_kernel(enc_ref, w1t_ref, b1_ref, w2t_ref, b2_ref, g_ref,
                    acc_ref, *, inv_hw):
    h = pl.program_id(1)

    @pl.when(h == 0)
    def _():
        acc_ref[...] = jnp.zeros_like(acc_ref)

    acc_ref[...] += jnp.sum(enc_ref[...], axis=-1)

    @pl.when(h == pl.num_programs(1) - 1)
    def _():
        pooled = acc_ref[...] * inv_hw
        z = jnp.maximum(
            jnp.dot(pooled, w1t_ref[...], preferred_element_type=jnp.float32)
            + b1_ref[...],
            0.0,
        )
        g_ref[...] = jax.nn.sigmoid(
            jnp.dot(z, w2t_ref[...], preferred_element_type=jnp.float32)
            + b2_ref[...]
        )[:, None, :]


def kernel(enc, dec, w1, b1, w2, b2):
    B, C, H, W = enc.shape
    Csq = w1.shape[0]
    HW = H * W

    enc2 = enc.reshape(B, C, HW)
    w1t = jnp.transpose(w1)
    w2t = jnp.transpose(w2)
    b1r = b1.reshape(1, Csq)
    b2r = b2.reshape(1, C)

    body = functools.partial(_se_gate_kernel, inv_hw=1.0 / HW)

    Bt, nhw = 2, 4
    HWt = HW // nhw
    g3 = pl.pallas_call(
        body,
        out_shape=jax.ShapeDtypeStruct((B, 1, C), jnp.float32),
        grid=(B // Bt, nhw),
        in_specs=[
            pl.BlockSpec((Bt, C, HWt), lambda b, h: (b, 0, h)),
            pl.BlockSpec((C, Csq), lambda b, h: (0, 0)),
            pl.BlockSpec((1, Csq), lambda b, h: (0, 0)),
            pl.BlockSpec((Csq, C), lambda b, h: (0, 0)),
            pl.BlockSpec((1, C), lambda b, h: (0, 0)),
        ],
        out_specs=pl.BlockSpec((Bt, 1, C), lambda b, h: (b, 0, 0)),
        scratch_shapes=[pltpu.VMEM((Bt, C), jnp.float32)],
        compiler_params=pltpu.CompilerParams(
            dimension_semantics=("parallel", "arbitrary"),
            vmem_limit_bytes=100 * 1024 * 1024,
        ),
    )(enc2, w1t, b1r, w2t, b2r)

    g = g3.reshape(B, C)
    out0 = jnp.pad(dec, ((0, 0), (0, C), (0, 0), (0, 0)))
    se = enc * g[:, :, None, None].astype(enc.dtype)
    return jax.lax.dynamic_update_slice(out0, se, (0, dec.shape[1], 0, 0))
